# baseline (speedup 1.0000x reference)
import functools

import jax
import jax.numpy as jnp
import numpy as np
from jax.experimental import pallas as pl
from jax.experimental.pallas import tpu as pltpu

_ROWS = 16384
_COLS = 1024
_TOT = _ROWS * _COLS
_BINS = 10
_LANES = 128
_FLAT_ROWS = _TOT // _LANES
_SLAB = 32
_BLOCK_ROWS = 8192


def _pointwise(p, t):
    u = jnp.exp(-jnp.abs(p))
    s = jnp.where(p >= 0.0, 1.0, u) / (1.0 + u)
    g = jnp.abs(s - t)
    loss = jnp.maximum(p, 0.0) - p * t + jnp.log1p(u)
    return g, loss


def _body(pred_ref, target_ref, out_ref, acc_ref):
    i = pl.program_id(0)
    nsteps = pl.num_programs(0)

    @pl.when(i == 0)
    def _init():
        for r in range(2):
            for b in range(_BINS):
                acc_ref[r, b] = jnp.float32(0.0)

    zero = jnp.zeros((8, _LANES), jnp.float32)
    thrs = [np.float32((b + 1) / _BINS) for b in range(_BINS - 1)]

    def step(k, carry):
        p = pred_ref[pl.ds(k * _SLAB, _SLAB), :]
        t = target_ref[pl.ds(k * _SLAB, _SLAB), :]
        g, loss = _pointwise(p, t)
        g3 = g.reshape(_SLAB // 8, 8, _LANES)
        l3 = loss.reshape(_SLAB // 8, 8, _LANES)
        cnts, sums, tl = carry
        new_cnts = []
        new_sums = []
        for b in range(_BINS - 1):
            m = g3 < thrs[b]
            new_cnts.append(cnts[b] + jnp.sum(jnp.where(m, 1.0, 0.0), axis=0))
            new_sums.append(sums[b] + jnp.sum(jnp.where(m, l3, 0.0), axis=0))
        tl = tl + jnp.sum(l3, axis=0)
        return new_cnts, new_sums, tl

    carry0 = ([zero] * (_BINS - 1), [zero] * (_BINS - 1), zero)
    cnts, sums, tl = jax.lax.fori_loop(0, _BLOCK_ROWS // _SLAB, step, carry0)

    for b in range(_BINS - 1):
        acc_ref[0, b] += jnp.sum(cnts[b])
        acc_ref[1, b] += jnp.sum(sums[b])
    acc_ref[1, _BINS - 1] += jnp.sum(tl)

    @pl.when(i == nsteps - 1)
    def _fin():
        tot = np.float32(_TOT)
        n = jnp.float32(0.0)
        acc = jnp.float32(0.0)
        prev_c = jnp.float32(0.0)
        prev_s = jnp.float32(0.0)
        for b in range(_BINS):
            c_cum = acc_ref[0, b] if b < _BINS - 1 else tot
            s_cum = acc_ref[1, b]
            c_b = c_cum - prev_c
            s_b = s_cum - prev_s
            nonempty = c_b > 0.0
            n += jnp.where(nonempty, 1.0, 0.0)
            acc += jnp.where(nonempty, s_b / jnp.where(nonempty, c_b, 1.0), 0.0)
            prev_c = c_cum
            prev_s = s_cum
        out_ref[0, 0] = acc / n


@functools.partial(jax.jit)
def kernel(pred, target):
    pred2 = pred.reshape(_FLAT_ROWS, _LANES)
    target2 = target.reshape(_FLAT_ROWS, _LANES)
    grid = (_FLAT_ROWS // _BLOCK_ROWS,)
    out = pl.pallas_call(
        _body,
        grid=grid,
        in_specs=[
            pl.BlockSpec((_BLOCK_ROWS, _LANES), lambda i: (i, 0)),
            pl.BlockSpec((_BLOCK_ROWS, _LANES), lambda i: (i, 0)),
        ],
        out_specs=pl.BlockSpec(memory_space=pltpu.SMEM),
        out_shape=jax.ShapeDtypeStruct((1, 1), jnp.float32),
        scratch_shapes=[pltpu.SMEM((2, _BINS), jnp.float32)],
        compiler_params=pltpu.CompilerParams(
            dimension_semantics=("arbitrary",),
        ),
    )(pred2, target2)
    return out[0, 0]


# speedup vs baseline: 1.1050x; 1.1050x over previous
import functools

import jax
import jax.numpy as jnp
import numpy as np
from jax.experimental import pallas as pl
from jax.experimental.pallas import tpu as pltpu

_ROWS = 16384
_COLS = 1024
_TOT = _ROWS * _COLS
_BINS = 10
_LANES = 128
_FLAT_ROWS = _TOT // _LANES
_SLAB = 64
_BLOCK_ROWS = 8192


def _pointwise(p, t):
    u = jnp.exp(-jnp.abs(p))
    s = jnp.where(p >= 0.0, 1.0, u) / (1.0 + u)
    g = jnp.abs(s - t)
    loss = jnp.maximum(p, 0.0) - p * t + jnp.log1p(u)
    return g, loss


def _body(pred_ref, target_ref, out_ref, acc_ref):
    i = pl.program_id(0)
    nsteps = pl.num_programs(0)

    @pl.when(i == 0)
    def _init():
        for r in range(2):
            for b in range(_BINS):
                acc_ref[r, b] = jnp.float32(0.0)

    zero = jnp.zeros((8, _LANES), jnp.float32)
    thrs = [np.float32((b + 1) / _BINS) for b in range(_BINS - 1)]

    def _load_pointwise(k):
        p = pred_ref[pl.ds(k * _SLAB, _SLAB), :]
        t = target_ref[pl.ds(k * _SLAB, _SLAB), :]
        g, loss = _pointwise(p, t)
        return (g.reshape(_SLAB // 8, 8, _LANES),
                loss.reshape(_SLAB // 8, 8, _LANES))

    def _bin(carry, g3, l3):
        cnts, sums, tl = carry
        new_cnts = []
        new_sums = []
        for b in range(_BINS - 1):
            m = g3 < thrs[b]
            new_cnts.append(cnts[b] + jnp.sum(jnp.where(m, 1.0, 0.0), axis=0))
            new_sums.append(sums[b] + jnp.sum(jnp.where(m, l3, 0.0), axis=0))
        tl = tl + jnp.sum(l3, axis=0)
        return new_cnts, new_sums, tl

    def step(k, carry):
        acc, (g_prev, l_prev) = carry
        nxt = _load_pointwise(k)
        return _bin(acc, g_prev, l_prev), nxt

    acc0 = ([zero] * (_BINS - 1), [zero] * (_BINS - 1), zero)
    acc, (g_last, l_last) = jax.lax.fori_loop(
        1, _BLOCK_ROWS // _SLAB, step, (acc0, _load_pointwise(0)))
    cnts, sums, tl = _bin(acc, g_last, l_last)

    for b in range(_BINS - 1):
        acc_ref[0, b] += jnp.sum(cnts[b])
        acc_ref[1, b] += jnp.sum(sums[b])
    acc_ref[1, _BINS - 1] += jnp.sum(tl)

    @pl.when(i == nsteps - 1)
    def _fin():
        tot = np.float32(_TOT)
        n = jnp.float32(0.0)
        acc = jnp.float32(0.0)
        prev_c = jnp.float32(0.0)
        prev_s = jnp.float32(0.0)
        for b in range(_BINS):
            c_cum = acc_ref[0, b] if b < _BINS - 1 else tot
            s_cum = acc_ref[1, b]
            c_b = c_cum - prev_c
            s_b = s_cum - prev_s
            nonempty = c_b > 0.0
            n += jnp.where(nonempty, 1.0, 0.0)
            acc += jnp.where(nonempty, s_b / jnp.where(nonempty, c_b, 1.0), 0.0)
            prev_c = c_cum
            prev_s = s_cum
        out_ref[0, 0] = acc / n


@functools.partial(jax.jit)
def kernel(pred, target):
    pred2 = pred.reshape(_FLAT_ROWS, _LANES)
    target2 = target.reshape(_FLAT_ROWS, _LANES)
    grid = (_FLAT_ROWS // _BLOCK_ROWS,)
    out = pl.pallas_call(
        _body,
        grid=grid,
        in_specs=[
            pl.BlockSpec((_BLOCK_ROWS, _LANES), lambda i: (i, 0)),
            pl.BlockSpec((_BLOCK_ROWS, _LANES), lambda i: (i, 0)),
        ],
        out_specs=pl.BlockSpec(memory_space=pltpu.SMEM),
        out_shape=jax.ShapeDtypeStruct((1, 1), jnp.float32),
        scratch_shapes=[pltpu.SMEM((2, _BINS), jnp.float32)],
        compiler_params=pltpu.CompilerParams(
            dimension_semantics=("arbitrary",),
        ),
    )(pred2, target2)
    return out[0, 0]


# speedup vs baseline: 1.5583x; 1.4102x over previous
import functools

import jax
import jax.numpy as jnp
import numpy as np
from jax.experimental import pallas as pl
from jax.experimental.pallas import tpu as pltpu

_ROWS = 16384
_COLS = 1024
_TOT = _ROWS * _COLS
_BINS = 10
_BLOCK_ROWS = 1024

_EDGE_LOGITS = [
    np.float32(np.log(np.float64(np.float32(b / 10.0))
                      / (1.0 - np.float64(np.float32(b / 10.0)))))
    for b in range(1, _BINS)
]


def _body(pred_ref, target_ref, out_ref, acc_ref):
    i = pl.program_id(0)
    nsteps = pl.num_programs(0)

    @pl.when(i == 0)
    def _init():
        for r in range(2):
            for b in range(_BINS):
                acc_ref[r, b] = jnp.float32(0.0)

    p = pred_ref[...]
    t = target_ref[...]
    u = jnp.exp(-jnp.abs(p))
    loss = jnp.maximum(p, 0.0) - p * t + jnp.log1p(u)
    q = jnp.where(t > 0.5, -p, p)

    for b in range(_BINS - 1):
        m = q < _EDGE_LOGITS[b]
        acc_ref[0, b] += jnp.sum(jnp.where(m, 1.0, 0.0))
        acc_ref[1, b] += jnp.sum(jnp.where(m, loss, 0.0))
    acc_ref[1, _BINS - 1] += jnp.sum(loss)

    @pl.when(i == nsteps - 1)
    def _fin():
        tot = np.float32(_TOT)
        n = jnp.float32(0.0)
        acc = jnp.float32(0.0)
        prev_c = jnp.float32(0.0)
        prev_s = jnp.float32(0.0)
        for b in range(_BINS):
            c_cum = acc_ref[0, b] if b < _BINS - 1 else tot
            s_cum = acc_ref[1, b]
            c_b = c_cum - prev_c
            s_b = s_cum - prev_s
            nonempty = c_b > 0.0
            n += jnp.where(nonempty, 1.0, 0.0)
            acc += jnp.where(nonempty, s_b / jnp.where(nonempty, c_b, 1.0), 0.0)
            prev_c = c_cum
            prev_s = s_cum
        out_ref[0, 0] = acc / n


@functools.partial(jax.jit)
def kernel(pred, target):
    grid = (_ROWS // _BLOCK_ROWS,)
    out = pl.pallas_call(
        _body,
        grid=grid,
        in_specs=[
            pl.BlockSpec((_BLOCK_ROWS, _COLS), lambda i: (i, 0)),
            pl.BlockSpec((_BLOCK_ROWS, _COLS), lambda i: (i, 0)),
        ],
        out_specs=pl.BlockSpec(memory_space=pltpu.SMEM),
        out_shape=jax.ShapeDtypeStruct((1, 1), jnp.float32),
        scratch_shapes=[pltpu.SMEM((2, _BINS), jnp.float32)],
        compiler_params=pltpu.CompilerParams(
            dimension_semantics=("arbitrary",),
        ),
    )(pred, target)
    return out[0, 0]
